# baseline (speedup 1.0000x reference)
import functools
import math

import jax
import jax.numpy as jnp
from jax import lax
from jax.experimental import pallas as pl
from jax.experimental.pallas import tpu as pltpu
from jax.experimental.pallas import tpu_sc as plsc

N = 10000
E = 320000
D_STR = 58
ALPHA = 0.4
THETA = 0.9

N_PAD = 10240
NTILES = 32
CHUNK = 128
G = 80
E_PAD = NTILES * G * CHUNK
ROWS_PER_TILE = N_PAD // 16
DH = 64
NBUF = 8
AHEAD = 4

_MESH = plsc.VectorSubcoreMesh(core_axis_name="c", subcore_axis_name="s")


@functools.partial(
    pl.kernel,
    out_type=jax.ShapeDtypeStruct((NTILES, N_PAD), jnp.float32),
    mesh=_MESH,
    compiler_params=pltpu.CompilerParams(needs_layout_passes=False),
    scratch_types=[
        pltpu.VMEM((G, CHUNK), jnp.int32),
        pltpu.VMEM((N_PAD,), jnp.float32),
    ],
)
def _deg_kernel(col_hbm, zeros1d_hbm, out_hbm, col_vm, acc_vm):
    c = lax.axis_index("c")
    s = lax.axis_index("s")
    w = c * 16 + s
    pltpu.sync_copy(zeros1d_hbm, acc_vm)
    pltpu.sync_copy(col_hbm.at[w], col_vm)
    ones = jnp.full((16,), 1.0, dtype=jnp.float32)

    def body(g, carry):
        for j in range(CHUNK // 16):
            idx = col_vm[g, pl.ds(j * 16, 16)]
            plsc.addupdate_scatter(acc_vm, [idx], ones)
        return carry

    lax.fori_loop(0, G, body, 0)
    pltpu.sync_copy(acc_vm, out_hbm.at[w])


@functools.partial(
    pl.kernel,
    out_type=jax.ShapeDtypeStruct((2, 2, N_PAD, DH), jnp.float32),
    mesh=_MESH,
    compiler_params=pltpu.CompilerParams(use_tc_tiling_on_sc=False),
    scratch_types=[
        pltpu.VMEM((G, CHUNK), jnp.int32),
        pltpu.VMEM((G, CHUNK), jnp.int32),
        *([pltpu.VMEM((CHUNK, DH), jnp.float32)] * NBUF),
        *([pltpu.SemaphoreType.DMA] * (2 * NBUF)),
        pltpu.VMEM_SHARED((N_PAD, DH), jnp.float32),
    ],
)
def _msg_kernel(hs_a_hbm, hs_b_hbm, row_hbm, col_hbm, zeros2d_hbm, out_hbm,
                row_vm, col_vm, *rest):
    bufs = rest[:NBUF]
    gsem = rest[NBUF:2 * NBUF]
    ssem = rest[2 * NBUF:3 * NBUF]
    acc_sh = rest[3 * NBUF]
    c = lax.axis_index("c")
    s = lax.axis_index("s")
    w = c * 16 + s
    my_rows = pl.ds(s * ROWS_PER_TILE, ROWS_PER_TILE)
    pltpu.sync_copy(zeros2d_hbm, acc_sh.at[my_rows])
    pltpu.sync_copy(row_hbm.at[w], row_vm)
    pltpu.sync_copy(col_hbm.at[w], col_vm)
    plsc.subcore_barrier()

    def run_phase(src_hbm, half):
        def start_gather(g, b):
            pltpu.async_copy(src_hbm.at[row_vm.at[g]], bufs[b], gsem[b])

        def start_scatter(g, b):
            pltpu.async_copy(bufs[b], acc_sh.at[col_vm.at[g]], ssem[b], add=True)

        def wait_gather(b):
            pltpu.make_async_copy(src_hbm.at[row_vm.at[0]], bufs[b], gsem[b]).wait()

        def wait_scatter(b):
            pltpu.make_async_copy(bufs[b], acc_sh.at[col_vm.at[0]], ssem[b]).wait()

        for b in range(AHEAD):
            start_gather(b, b)
        for b in range(NBUF):
            wait_gather(b)
            start_scatter(b, b)
            f = b + AHEAD
            bf = f % NBUF
            if f >= NBUF:
                wait_scatter(bf)
            start_gather(f, bf)

        def body(o, carry):
            for b in range(NBUF):
                g = o * NBUF + b
                wait_gather(b)
                start_scatter(g, b)
                bf = (b + AHEAD) % NBUF
                wait_scatter(bf)
                start_gather(g + AHEAD, bf)
            return carry

        lax.fori_loop(1, (G // NBUF) - 1, body, 0)

        for b in range(NBUF):
            g = G - NBUF + b
            wait_gather(b)
            start_scatter(g, b)
            f = g + AHEAD
            if f < G:
                bf = f % NBUF
                wait_scatter(bf)
                start_gather(f, bf)
        for b in range(NBUF):
            wait_scatter(b)

        plsc.subcore_barrier()
        pltpu.sync_copy(acc_sh.at[my_rows], out_hbm.at[c, half, my_rows])

    run_phase(hs_a_hbm, 0)
    pltpu.sync_copy(zeros2d_hbm, acc_sh.at[my_rows])
    plsc.subcore_barrier()
    run_phase(hs_b_hbm, 1)


_BLK = 2048
_GRID = N_PAD // _BLK
_DOT = dict(preferred_element_type=jnp.float32, precision=jax.lax.Precision.HIGHEST)


def _pre_body(xcat_ref, degp_ref, wblk_ref, bcat_ref,
              h0_ref, hsa_ref, hsb_ref, dis_ref):
    deg = jnp.sum(degp_ref[...], axis=0)
    dis = jnp.where(deg > 0, lax.rsqrt(jnp.maximum(deg, 1e-12)), 0.0)
    h = jnp.maximum(jnp.dot(xcat_ref[...], wblk_ref[...], **_DOT) + bcat_ref[...], 0.0)
    h0_ref[...] = h
    hs = h * dis[:, None]
    hsa_ref[...] = hs[:, :DH]
    hsb_ref[...] = hs[:, DH:]
    dis_ref[...] = jnp.broadcast_to(dis[:, None], (_BLK, 128))


_pre_call = pl.pallas_call(
    _pre_body,
    grid=(_GRID,),
    in_specs=[
        pl.BlockSpec((_BLK, 192), lambda i: (i, 0)),
        pl.BlockSpec((NTILES, _BLK), lambda i: (0, i)),
        pl.BlockSpec((192, 128), lambda i: (0, 0)),
        pl.BlockSpec((1, 128), lambda i: (0, 0)),
    ],
    out_specs=[
        pl.BlockSpec((_BLK, 128), lambda i: (i, 0)),
        pl.BlockSpec((_BLK, DH), lambda i: (i, 0)),
        pl.BlockSpec((_BLK, DH), lambda i: (i, 0)),
        pl.BlockSpec((_BLK, 128), lambda i: (i, 0)),
    ],
    out_shape=[
        jax.ShapeDtypeStruct((N_PAD, 128), jnp.float32),
        jax.ShapeDtypeStruct((N_PAD, DH), jnp.float32),
        jax.ShapeDtypeStruct((N_PAD, DH), jnp.float32),
        jax.ShapeDtypeStruct((N_PAD, 128), jnp.float32),
    ],
)


def _mix(agg_ref, h0_ref, dis_ref, w1_ref, w2_ref, beta):
    aggsum = jnp.concatenate(
        [agg_ref[0, 0] + agg_ref[1, 0], agg_ref[0, 1] + agg_ref[1, 1]], axis=1)
    agg = aggsum * dis_ref[...] * (1.0 - ALPHA)
    h0a = h0_ref[...] * ALPHA
    t = (1.0 - beta) * (agg + h0a) + beta * (
        jnp.dot(agg, w1_ref[...], **_DOT) + jnp.dot(h0a, w2_ref[...], **_DOT))
    return jnp.maximum(t, 0.0)


def _layer_body(agg_ref, h0_ref, dis_ref, w1_ref, w2_ref,
                hsa_ref, hsb_ref, *, beta):
    hs = _mix(agg_ref, h0_ref, dis_ref, w1_ref, w2_ref, beta) * dis_ref[...]
    hsa_ref[...] = hs[:, :DH]
    hsb_ref[...] = hs[:, DH:]


def _final_body(agg_ref, h0_ref, dis_ref, w1_ref, w2_ref, wz_ref, bz_ref,
                out_ref, *, beta):
    h = _mix(agg_ref, h0_ref, dis_ref, w1_ref, w2_ref, beta)
    out_ref[...] = jnp.dot(h, wz_ref[...], **_DOT) + bz_ref[...]


_COMMON_SPECS = [
    pl.BlockSpec((2, 2, _BLK, DH), lambda i: (0, 0, i, 0)),
    pl.BlockSpec((_BLK, 128), lambda i: (i, 0)),
    pl.BlockSpec((_BLK, 128), lambda i: (i, 0)),
    pl.BlockSpec((128, 128), lambda i: (0, 0)),
    pl.BlockSpec((128, 128), lambda i: (0, 0)),
]


def _beta(i):
    return float(math.log(THETA / (i + 1) + 1.0))


_layer1_call = pl.pallas_call(
    functools.partial(_layer_body, beta=_beta(0)),
    grid=(_GRID,),
    in_specs=_COMMON_SPECS,
    out_specs=[pl.BlockSpec((_BLK, DH), lambda i: (i, 0))] * 2,
    out_shape=[jax.ShapeDtypeStruct((N_PAD, DH), jnp.float32)] * 2,
)

_final_call = pl.pallas_call(
    functools.partial(_final_body, beta=_beta(1)),
    grid=(_GRID,),
    in_specs=_COMMON_SPECS + [
        pl.BlockSpec((128, 128), lambda i: (0, 0)),
        pl.BlockSpec((1, 128), lambda i: (0, 0)),
    ],
    out_specs=pl.BlockSpec((_BLK, 128), lambda i: (i, 0)),
    out_shape=jax.ShapeDtypeStruct((N_PAD, 128), jnp.float32),
)


def kernel(x, data_str, edge_index, lins0_w, lins0_b, lins1_w, lins1_b,
           lin11_w, lin11_b, lin3_w, lin3_b,
           convs_w1, convs_w2, convs1_w1, convs1_w2):
    f32 = jnp.float32
    row = edge_index[0]
    col = edge_index[1]
    row_p = jnp.concatenate([row, jnp.zeros((E_PAD - E,), jnp.int32)])
    col_p = jnp.concatenate([col, jnp.full((E_PAD - E,), N, jnp.int32)])
    row_t = row_p.reshape(NTILES, G, CHUNK)
    col_t = col_p.reshape(NTILES, G, CHUNK)

    x_p = jnp.pad(x, ((0, N_PAD - N), (0, 0)))
    ds_p = jnp.pad(data_str, ((0, N_PAD - N), (0, 64 - D_STR)))
    xcat = jnp.concatenate([x_p, ds_p], axis=1)

    zero64 = jnp.zeros((64, 64), f32)
    wblk = jnp.concatenate([
        jnp.concatenate([lins0_w, jnp.zeros((128, 64), f32)], axis=1),
        jnp.concatenate([jnp.zeros((64, 64), f32),
                         jnp.pad(lin11_w, ((0, 64 - D_STR), (0, 0)))], axis=1),
    ], axis=0)
    bcat = jnp.concatenate([lins0_b, lin11_b]).reshape(1, 128)

    def blkdiag(a, b):
        return jnp.concatenate([
            jnp.concatenate([a, zero64], axis=1),
            jnp.concatenate([zero64, b], axis=1),
        ], axis=0)

    w1b = [blkdiag(convs_w1[i], convs1_w1[i]) for i in range(2)]
    w2b = [blkdiag(convs_w2[i], convs1_w2[i]) for i in range(2)]

    wz = jnp.zeros((128, 128), f32)
    wz = wz.at[:64, 0:1].set(lins1_w)
    wz = wz.at[64:, 1:2].set(lin3_w)
    bz = jnp.zeros((1, 128), f32)
    bz = bz.at[0, 0].set(lins1_b[0])
    bz = bz.at[0, 1].set(lin3_b[0])

    zeros1d = jnp.zeros((N_PAD,), f32)
    zeros2d = jnp.zeros((ROWS_PER_TILE, DH), f32)

    degp = _deg_kernel(col_t, zeros1d)
    h0, hsa, hsb, dis = _pre_call(xcat, degp, wblk, bcat)

    agg1 = _msg_kernel(hsa, hsb, row_t, col_t, zeros2d)
    hsa1, hsb1 = _layer1_call(agg1, h0, dis, w1b[0], w2b[0])

    agg2 = _msg_kernel(hsa1, hsb1, row_t, col_t, zeros2d)
    zcat = _final_call(agg2, h0, dis, w1b[1], w2b[1], wz, bz)

    return (zcat[:N, 0:1], zcat[:N, 1:2])
